# Initial kernel scaffold; baseline (speedup 1.0000x reference)
import jax
import jax.numpy as jnp
from jax.experimental import pallas as pl


def kernel(x0, x1, edge_index0, edge_index1, W_rbf0, b_rbf0, W_rbf1, b_rbf1, Wself1, Wneigh1, bconv1, Wself2, Wneigh2, bconv2, W_fc1, b_fc1, W_out, b_out, W_read, b_read):
    raise NotImplementedError("write your pallas kernel here")


# speedup vs baseline: 50.1211x; 50.1211x over previous
import functools

import jax
import jax.numpy as jnp
from jax import lax
from jax.experimental import pallas as pl
from jax.experimental.pallas import tpu as pltpu
from jax.experimental.pallas import tpu_sc as plsc

N = 10000
E = 320000
D = 128
H = 128
O = 64
L = 3

N_PAD = 10240
SINK = N_PAD - 1
N_TILES = 16
EDG_PER_TILE = E // N_TILES
CHUNK = 128
N_CHUNKS = (EDG_PER_TILE + CHUNK - 1) // CHUNK
EDG_PAD = N_CHUNKS * CHUNK
SLICE = N_PAD // N_TILES

XBLK = 1280
N_XBLK = N_PAD // XBLK


def _sc_body(src_hbm, dst_hbm, u_hbm, src_v, dst_v, vals_v, ones_v, zeros_v,
             tmp_v, dinv_v, r_v, r_s, acc_s, sem_g, sem_s):
    c = lax.axis_index("c")
    s = lax.axis_index("s")
    off = s * SLICE

    for i in range(CHUNK // 16):
        ones_v[pl.ds(i * 16, 16)] = jnp.full((16,), 1.0, jnp.float32)
    for i in range(SLICE // 16):
        zeros_v[pl.ds(i * 16, 16)] = jnp.zeros((16,), jnp.float32)

    pltpu.sync_copy(src_hbm.at[c, s], src_v)
    pltpu.sync_copy(dst_hbm.at[c, s], dst_v)
    pltpu.sync_copy(zeros_v, acc_s.at[pl.ds(off, SLICE)])
    plsc.subcore_barrier()

    def fire_deg(j, carry):
        pltpu.async_copy(ones_v, acc_s.at[dst_v.at[j]], sem_s, add=True)
        return carry

    def drain_deg(j, carry):
        pltpu.make_async_copy(ones_v, acc_s.at[dst_v.at[j]], sem_s).wait()
        return carry

    lax.fori_loop(0, N_CHUNKS, fire_deg, 0)
    lax.fori_loop(0, N_CHUNKS, drain_deg, 0)
    plsc.subcore_barrier()

    pltpu.sync_copy(acc_s.at[pl.ds(off, SLICE)], tmp_v)
    for i in range(SLICE // 16):
        sl = pl.ds(i * 16, 16)
        deg = tmp_v[sl]
        dinv = 1.0 / jnp.maximum(deg, 1.0)
        dinv_v[sl] = dinv
        gidx = off + i * 16 + lax.iota(jnp.int32, 16)
        u0 = jnp.where(gidx < N, 1.0, 0.0).astype(jnp.float32)
        tmp_v[sl] = u0
        r_v[sl] = u0 * dinv
    pltpu.sync_copy(tmp_v, u_hbm.at[c, 0, pl.ds(off, SLICE)])
    pltpu.sync_copy(r_v, r_s.at[pl.ds(off, SLICE)])
    pltpu.sync_copy(zeros_v, acc_s.at[pl.ds(off, SLICE)])
    plsc.subcore_barrier()

    for p in (1, 2, 3):
        def fire_g(j, carry):
            pltpu.async_copy(r_s.at[dst_v.at[j]], vals_v.at[j], sem_g)
            return carry

        def drain_g(j, carry):
            pltpu.make_async_copy(r_s.at[dst_v.at[j]], vals_v.at[j], sem_g).wait()
            return carry

        def fire_s(j, carry):
            pltpu.async_copy(vals_v.at[j], acc_s.at[src_v.at[j]], sem_s, add=True)
            return carry

        def drain_s(j, carry):
            pltpu.make_async_copy(vals_v.at[j], acc_s.at[src_v.at[j]], sem_s).wait()
            return carry

        lax.fori_loop(0, N_CHUNKS, fire_g, 0)
        lax.fori_loop(0, N_CHUNKS, drain_g, 0)
        lax.fori_loop(0, N_CHUNKS, fire_s, 0)
        lax.fori_loop(0, N_CHUNKS, drain_s, 0)
        plsc.subcore_barrier()

        pltpu.sync_copy(acc_s.at[pl.ds(off, SLICE)], tmp_v)
        pltpu.sync_copy(tmp_v, u_hbm.at[c, p, pl.ds(off, SLICE)])
        if p < 3:
            for i in range(SLICE // 16):
                sl = pl.ds(i * 16, 16)
                r_v[sl] = tmp_v[sl] * dinv_v[sl]
            pltpu.sync_copy(r_v, r_s.at[pl.ds(off, SLICE)])
            pltpu.sync_copy(zeros_v, acc_s.at[pl.ds(off, SLICE)])
            plsc.subcore_barrier()


@functools.cache
def _sc_adjoint():
    return pl.kernel(
        _sc_body,
        out_type=jax.ShapeDtypeStruct((2, 8, N_PAD), jnp.float32),
        mesh=plsc.VectorSubcoreMesh(core_axis_name="c", subcore_axis_name="s"),
        scratch_types=[
        pltpu.VMEM((N_CHUNKS, CHUNK), jnp.int32),
        pltpu.VMEM((N_CHUNKS, CHUNK), jnp.int32),
        pltpu.VMEM((N_CHUNKS, CHUNK), jnp.float32),
        pltpu.VMEM((CHUNK,), jnp.float32),
        pltpu.VMEM((SLICE,), jnp.float32),
        pltpu.VMEM((SLICE,), jnp.float32),
        pltpu.VMEM((SLICE,), jnp.float32),
        pltpu.VMEM((SLICE,), jnp.float32),
        pltpu.VMEM_SHARED((N_PAD,), jnp.float32),
        pltpu.VMEM_SHARED((N_PAD,), jnp.float32),
            pltpu.SemaphoreType.DMA,
            pltpu.SemaphoreType.DMA,
        ],
    )


def _tc_body(x0_ref, x1_ref, wr0_ref, wr1_ref, br0_ref, br1_ref, u_ref,
             ws0_ref, wn0_ref, bc0_ref, ws1_ref, wn1_ref, bc1_ref,
             wfc_ref, bfc_ref, wout_ref, bout_ref, wreadt_ref, bread_ref,
             out_ref, m0_acc, m1_acc, s0_acc, s1_acc):
    b = pl.program_id(0)

    @pl.when(b == 0)
    def _init():
        m0_acc[...] = jnp.zeros_like(m0_acc)
        m1_acc[...] = jnp.zeros_like(m1_acc)
        s0_acc[...] = jnp.zeros_like(s0_acc)
        s1_acc[...] = jnp.zeros_like(s1_acc)

    scale = jnp.float32(jnp.sqrt(2.0 / H))
    h0 = jnp.cos(jnp.dot(x0_ref[...], wr0_ref[...],
                         preferred_element_type=jnp.float32) + br0_ref[...]) * scale
    h1 = jnp.cos(jnp.dot(x1_ref[...], wr1_ref[...],
                         preferred_element_type=jnp.float32) + br1_ref[...]) * scale
    u0b = u_ref[0]
    u1b = u_ref[1]
    m0_acc[...] += jnp.dot(u0b, h0, preferred_element_type=jnp.float32)
    m1_acc[...] += jnp.dot(u1b, h1, preferred_element_type=jnp.float32)
    s0_acc[...] += jnp.broadcast_to(jnp.sum(u0b, axis=1, keepdims=True), (8, H))
    s1_acc[...] += jnp.broadcast_to(jnp.sum(u1b, axis=1, keepdims=True), (8, H))

    @pl.when(b == N_XBLK - 1)
    def _final():
        def recurse(m_acc, s_acc, ws_ref, wn_ref, bc_ref):
            q = [m_acc[k:k + 1, :] for k in range(L + 1)]
            sr = [s_acc[k:k + 1, :] for k in range(L + 1)]
            for i in range(L):
                q = [jnp.dot(q[k], ws_ref[i], preferred_element_type=jnp.float32)
                     + jnp.dot(q[k + 1], wn_ref[i], preferred_element_type=jnp.float32)
                     + sr[k] * bc_ref[i]
                     for k in range(L - i)]
            return q[0]

        inv_n = jnp.float32(1.0 / N)
        y2 = recurse(m0_acc, s0_acc, ws0_ref, wn0_ref, bc0_ref) * inv_n
        y1 = recurse(m1_acc, s1_acc, ws1_ref, wn1_ref, bc1_ref) * inv_n
        y = jnp.concatenate([y2, y1], axis=1)
        h = jnp.dot(y, wfc_ref[...], preferred_element_type=jnp.float32) + bfc_ref[...]
        h = jnp.maximum(h, 0.0)
        g = jnp.dot(h, wout_ref[...], preferred_element_type=jnp.float32) + bout_ref[...]
        g = jnp.where(g > 0, g, 0.01 * g)
        o = jnp.sum(g * wreadt_ref[...], axis=1, keepdims=True) + bread_ref[...]
        out_ref[...] = jnp.broadcast_to(o, (1, H))


def _full(shape):
    return pl.BlockSpec(shape, lambda b: tuple(0 for _ in shape))


_tc_dense = pl.pallas_call(
    _tc_body,
    grid=(N_XBLK,),
    in_specs=[
        pl.BlockSpec((XBLK, D), lambda b: (b, 0)),
        pl.BlockSpec((XBLK, D), lambda b: (b, 0)),
        _full((D, H)), _full((D, H)),
        _full((1, H)), _full((1, H)),
        pl.BlockSpec((2, 8, XBLK), lambda b: (0, 0, b)),
        _full((L, H, H)), _full((L, H, H)), _full((L, 1, H)),
        _full((L, H, H)), _full((L, H, H)), _full((L, 1, H)),
        _full((2 * H, H)), _full((1, H)),
        _full((H, O)), _full((1, O)),
        _full((1, O)), _full((1, 1)),
    ],
    out_specs=pl.BlockSpec((1, H), lambda b: (0, 0)),
    out_shape=jax.ShapeDtypeStruct((1, H), jnp.float32),
    scratch_shapes=[
        pltpu.VMEM((8, H), jnp.float32),
        pltpu.VMEM((8, H), jnp.float32),
        pltpu.VMEM((8, H), jnp.float32),
        pltpu.VMEM((8, H), jnp.float32),
    ],
)


def _prep_edges(ei):
    a = ei.reshape(2, N_TILES, EDG_PER_TILE)
    pad = jnp.full((2, N_TILES, EDG_PAD - EDG_PER_TILE), SINK, dtype=jnp.int32)
    return jnp.concatenate([a, pad], axis=2).reshape(2, N_TILES, N_CHUNKS, CHUNK)


def kernel(x0, x1, edge_index0, edge_index1, W_rbf0, b_rbf0, W_rbf1, b_rbf1,
           Wself1, Wneigh1, bconv1, Wself2, Wneigh2, bconv2,
           W_fc1, b_fc1, W_out, b_out, W_read, b_read):
    e0 = _prep_edges(edge_index0)
    e1 = _prep_edges(edge_index1)
    src_all = jnp.stack([e0[0], e1[0]])
    dst_all = jnp.stack([e0[1], e1[1]])

    U = _sc_adjoint()(src_all, dst_all)

    x0p = jnp.pad(x0, ((0, N_PAD - N), (0, 0)))
    x1p = jnp.pad(x1, ((0, N_PAD - N), (0, 0)))

    out = _tc_dense(
        x0p, x1p, W_rbf0, W_rbf1,
        b_rbf0.reshape(1, H), b_rbf1.reshape(1, H), U,
        Wself2, Wneigh2, bconv2.reshape(L, 1, H),
        Wself1, Wneigh1, bconv1.reshape(L, 1, H),
        W_fc1, b_fc1.reshape(1, H), W_out, b_out.reshape(1, O),
        W_read.reshape(1, O), b_read.reshape(1, 1),
    )
    return out[0, 0:1]


# speedup vs baseline: 73.1978x; 1.4604x over previous
import functools

import jax
import jax.numpy as jnp
from jax import lax
from jax.experimental import pallas as pl
from jax.experimental.pallas import tpu as pltpu
from jax.experimental.pallas import tpu_sc as plsc

N = 10000
E = 320000
D = 128
H = 128
O = 64
L = 3

N_PAD = 10240
SINK = N_PAD - 1
N_TILES = 16
CHUNK = 128
TOT_CHUNKS = E // CHUNK
N_CHUNKS = -(-TOT_CHUNKS // N_TILES)
SLICE = N_PAD // N_TILES

XBLK = 1000
N_XBLK = N // XBLK


def _sc_body(ei0, ei1, u_hbm, src_v, dst_v, vals_v, ones_v, zeros_v,
             tmp_v, dinv_v, r_v, r_s, acc_s, sem_g, sem_s):
    c = lax.axis_index("c")
    s = lax.axis_index("s")
    off = s * SLICE
    nc = N_CHUNKS

    for i in range(CHUNK // 16):
        ones_v[pl.ds(i * 16, 16)] = jnp.full((16,), 1.0, jnp.float32)
    for i in range(SLICE // 16):
        zeros_v[pl.ds(i * 16, 16)] = jnp.zeros((16,), jnp.float32)

    def load_slabs(ei):
        pltpu.sync_copy(ei.at[0, s], src_v)
        pltpu.sync_copy(ei.at[1, s], dst_v)

    @pl.when(c == 0)
    def _():
        load_slabs(ei0)

    @pl.when(c == 1)
    def _():
        load_slabs(ei1)

    pltpu.sync_copy(zeros_v, acc_s.at[pl.ds(off, SLICE)])
    plsc.subcore_barrier()

    def fire_deg(j, carry):
        pltpu.async_copy(ones_v, acc_s.at[dst_v.at[j]], sem_s, add=True)
        return carry

    def drain_deg(j, carry):
        pltpu.make_async_copy(ones_v, acc_s.at[dst_v.at[j]], sem_s).wait()
        return carry

    lax.fori_loop(0, nc, fire_deg, 0)
    lax.fori_loop(0, nc, drain_deg, 0)
    plsc.subcore_barrier()

    pltpu.sync_copy(acc_s.at[pl.ds(off, SLICE)], tmp_v)
    for i in range(SLICE // 16):
        sl = pl.ds(i * 16, 16)
        deg = tmp_v[sl]
        dinv = 1.0 / jnp.maximum(deg, 1.0)
        dinv_v[sl] = dinv
        gidx = off + i * 16 + lax.iota(jnp.int32, 16)
        u0 = jnp.where(gidx < N, 1.0, 0.0).astype(jnp.float32)
        tmp_v[sl] = u0
        r_v[sl] = u0 * dinv
    pltpu.sync_copy(tmp_v, u_hbm.at[c, 0, pl.ds(off, SLICE)])
    pltpu.sync_copy(r_v, r_s.at[pl.ds(off, SLICE)])
    pltpu.sync_copy(zeros_v, acc_s.at[pl.ds(off, SLICE)])
    plsc.subcore_barrier()

    for p in (1, 2, 3):
        def fire_g(j, carry):
            pltpu.async_copy(r_s.at[dst_v.at[j]], vals_v.at[j], sem_g)
            return carry

        def drain_g(j, carry):
            pltpu.make_async_copy(r_s.at[dst_v.at[j]], vals_v.at[j], sem_g).wait()
            return carry

        def fire_s(j, carry):
            pltpu.async_copy(vals_v.at[j], acc_s.at[src_v.at[j]], sem_s, add=True)
            return carry

        def drain_s(j, carry):
            pltpu.make_async_copy(vals_v.at[j], acc_s.at[src_v.at[j]], sem_s).wait()
            return carry

        lax.fori_loop(0, nc, fire_g, 0)
        lax.fori_loop(0, nc, drain_g, 0)
        lax.fori_loop(0, nc, fire_s, 0)
        lax.fori_loop(0, nc, drain_s, 0)
        plsc.subcore_barrier()

        pltpu.sync_copy(acc_s.at[pl.ds(off, SLICE)], tmp_v)
        pltpu.sync_copy(tmp_v, u_hbm.at[c, p, pl.ds(off, SLICE)])
        if p < 3:
            for i in range(SLICE // 16):
                sl = pl.ds(i * 16, 16)
                r_v[sl] = tmp_v[sl] * dinv_v[sl]
            pltpu.sync_copy(r_v, r_s.at[pl.ds(off, SLICE)])
            pltpu.sync_copy(zeros_v, acc_s.at[pl.ds(off, SLICE)])
            plsc.subcore_barrier()


@functools.cache
def _sc_adjoint():
    return pl.kernel(
        _sc_body,
        out_type=jax.ShapeDtypeStruct((2, 8, N_PAD), jnp.float32),
        mesh=plsc.VectorSubcoreMesh(core_axis_name="c", subcore_axis_name="s"),
        scratch_types=[
            pltpu.VMEM((N_CHUNKS, CHUNK), jnp.int32),
            pltpu.VMEM((N_CHUNKS, CHUNK), jnp.int32),
            pltpu.VMEM((N_CHUNKS, CHUNK), jnp.float32),
            pltpu.VMEM((CHUNK,), jnp.float32),
            pltpu.VMEM((SLICE,), jnp.float32),
            pltpu.VMEM((SLICE,), jnp.float32),
            pltpu.VMEM((SLICE,), jnp.float32),
            pltpu.VMEM((SLICE,), jnp.float32),
            pltpu.VMEM_SHARED((N_PAD,), jnp.float32),
            pltpu.VMEM_SHARED((N_PAD,), jnp.float32),
            pltpu.SemaphoreType.DMA,
            pltpu.SemaphoreType.DMA,
        ],
    )


def _rbf_body(x0_ref, x1_ref, wr0_ref, wr1_ref, br0_ref, br1_ref,
              h0_ref, h1_ref):
    scale = jnp.float32(jnp.sqrt(2.0 / H))
    h0_ref[...] = jnp.cos(jnp.dot(x0_ref[...], wr0_ref[...],
                                  preferred_element_type=jnp.float32)
                          + br0_ref[...]) * scale
    h1_ref[...] = jnp.cos(jnp.dot(x1_ref[...], wr1_ref[...],
                                  preferred_element_type=jnp.float32)
                          + br1_ref[...]) * scale


def _full(shape):
    return pl.BlockSpec(shape, lambda b: tuple(0 for _ in shape))


_tc_rbf = pl.pallas_call(
    _rbf_body,
    grid=(N_XBLK,),
    in_specs=[
        pl.BlockSpec((XBLK, D), lambda b: (b, 0)),
        pl.BlockSpec((XBLK, D), lambda b: (b, 0)),
        _full((D, H)), _full((D, H)),
        _full((1, H)), _full((1, H)),
    ],
    out_specs=[
        pl.BlockSpec((XBLK, H), lambda b: (b, 0)),
        pl.BlockSpec((XBLK, H), lambda b: (b, 0)),
    ],
    out_shape=[
        jax.ShapeDtypeStruct((N, H), jnp.float32),
        jax.ShapeDtypeStruct((N, H), jnp.float32),
    ],
)


def _head_body(h0_ref, h1_ref, u_ref,
               ws0_ref, wn0_ref, bc0_ref, ws1_ref, wn1_ref, bc1_ref,
               wfc_ref, bfc_ref, wout_ref, bout_ref, wreadt_ref, bread_ref,
               out_ref):
    zpad = jnp.zeros((N_PAD - N, H), jnp.float32)

    def moments(h_ref, g):
        hp = jnp.concatenate([h_ref[...], zpad], axis=0)
        u = u_ref[g]
        m = jnp.dot(u, hp, preferred_element_type=jnp.float32)
        s = jnp.broadcast_to(jnp.sum(u, axis=1, keepdims=True), (8, H))
        return m, s

    m0, s0 = moments(h0_ref, 0)
    m1, s1 = moments(h1_ref, 1)

    def recurse(m, s, ws_ref, wn_ref, bc_ref):
        q = [m[k:k + 1, :] for k in range(L + 1)]
        sr = [s[k:k + 1, :] for k in range(L + 1)]
        for i in range(L):
            q = [jnp.dot(q[k], ws_ref[i], preferred_element_type=jnp.float32)
                 + jnp.dot(q[k + 1], wn_ref[i], preferred_element_type=jnp.float32)
                 + sr[k] * bc_ref[i]
                 for k in range(L - i)]
        return q[0]

    inv_n = jnp.float32(1.0 / N)
    y2 = recurse(m0, s0, ws0_ref, wn0_ref, bc0_ref) * inv_n
    y1 = recurse(m1, s1, ws1_ref, wn1_ref, bc1_ref) * inv_n
    y = jnp.concatenate([y2, y1], axis=1)
    h = jnp.dot(y, wfc_ref[...], preferred_element_type=jnp.float32) + bfc_ref[...]
    h = jnp.maximum(h, 0.0)
    g = jnp.dot(h, wout_ref[...], preferred_element_type=jnp.float32) + bout_ref[...]
    g = jnp.where(g > 0, g, 0.01 * g)
    o = jnp.sum(g * wreadt_ref[...], axis=1, keepdims=True) + bread_ref[...]
    out_ref[...] = jnp.broadcast_to(o, (1, H))


_tc_head = pl.pallas_call(
    _head_body,
    out_shape=jax.ShapeDtypeStruct((1, H), jnp.float32),
)


def kernel(x0, x1, edge_index0, edge_index1, W_rbf0, b_rbf0, W_rbf1, b_rbf1,
           Wself1, Wneigh1, bconv1, Wself2, Wneigh2, bconv2,
           W_fc1, b_fc1, W_out, b_out, W_read, b_read):
    def prep(ei):
        a = ei.reshape(2, TOT_CHUNKS, CHUNK)
        a = jnp.pad(a, ((0, 0), (0, N_TILES * N_CHUNKS - TOT_CHUNKS), (0, 0)),
                    constant_values=SINK)
        return a.reshape(2, N_TILES, N_CHUNKS, CHUNK)

    U = _sc_adjoint()(prep(edge_index0), prep(edge_index1))
    h0, h1 = _tc_rbf(x0, x1, W_rbf0, W_rbf1,
                     b_rbf0.reshape(1, H), b_rbf1.reshape(1, H))

    out = _tc_head(
        h0, h1, U,
        Wself2, Wneigh2, bconv2.reshape(L, 1, H),
        Wself1, Wneigh1, bconv1.reshape(L, 1, H),
        W_fc1, b_fc1.reshape(1, H), W_out, b_out.reshape(1, O),
        W_read.reshape(1, O), b_read.reshape(1, 1),
    )
    return out[0, 0:1]
